# baseline (speedup 1.0000x reference)
import functools

import jax
import jax.numpy as jnp
from jax import lax
from jax.experimental import pallas as pl
from jax.experimental.pallas import tpu as pltpu
from jax.experimental.pallas import tpu_sc as plsc

N = 1000
RPAD = 1024
B = 1024
SEQ = 20
L = 16
NW = 32
BPW = 16
CH = 2 * SEQ


def _recip_body(x_ref, o_ref):
    x = x_ref[...]
    d = jnp.where(x == 0.0, jnp.float32(9999999.99), x)
    r = 1.0 / d
    o_ref[...] = jnp.concatenate(
        [r, jnp.full((N, RPAD - N), 1.0, jnp.float32)], axis=1)


_recip_call = pl.pallas_call(
    _recip_body,
    out_shape=jax.ShapeDtypeStruct((N, RPAD), jnp.float32),
)


_sc_mesh = plsc.VectorSubcoreMesh(core_axis_name="c", subcore_axis_name="s")


@functools.partial(
    pl.kernel,
    mesh=_sc_mesh,
    out_type=jax.ShapeDtypeStruct((B // 2, CH, RPAD), jnp.float32),
    compiler_params=pltpu.CompilerParams(
        use_tc_tiling_on_sc=True, needs_layout_passes=False),
    scratch_types=[
        pltpu.VMEM((N,), jnp.int32),
        pltpu.VMEM((2 * BPW * SEQ,), jnp.int32),
        pltpu.VMEM((BPW, CH), jnp.int32),
        pltpu.VMEM((3, CH, RPAD), jnp.float32),
        pltpu.SemaphoreType.DMA,
        pltpu.SemaphoreType.DMA,
        pltpu.SemaphoreType.DMA,
        pltpu.SemaphoreType.DMA,
        pltpu.SemaphoreType.DMA,
        pltpu.SemaphoreType.DMA,
    ],
)
def _sc_gather(venue_hbm, poi_hbm, table_hbm, out_hbm,
               venue_v, poi_v, idx_v, rows_v,
               sem_g0, sem_g1, sem_g2, sem_o0, sem_o1, sem_o2):
    wid = lax.axis_index("s") * 2 + lax.axis_index("c")
    sem_g = (sem_g0, sem_g1, sem_g2)
    sem_o = (sem_o0, sem_o1, sem_o2)
    half = BPW * SEQ

    pltpu.sync_copy(venue_hbm, venue_v)
    pltpu.sync_copy(poi_hbm.at[pl.ds(wid * half, half)],
                    poi_v.at[pl.ds(0, half)])
    pltpu.sync_copy(poi_hbm.at[pl.ds((B // 2) * SEQ + wid * half, half)],
                    poi_v.at[pl.ds(half, half)])

    iota = lax.iota(jnp.int32, L)
    for c in range(BPW):
        base_a = c * SEQ
        base_b = half + c * SEQ
        p0 = iota + base_a
        p1 = jnp.where(iota + 16 < SEQ, iota + (base_a + 16),
                       iota + (base_b - 4))
        p2 = iota + (base_b + 4)
        for off, pos in ((0, p0), (16, p1), (24, p2)):
            v = plsc.load_gather(poi_v, [pos])
            idx_v[c, pl.ds(off, L)] = plsc.load_gather(venue_v, [v])

    def start_gather(c):
        pltpu.async_copy(table_hbm.at[idx_v.at[c]],
                         rows_v.at[c % 3], sem_g[c % 3])

    def wait_gather(c):
        pltpu.make_async_copy(table_hbm.at[idx_v.at[c]],
                              rows_v.at[c % 3], sem_g[c % 3]).wait()

    def start_out(c):
        pltpu.async_copy(rows_v.at[c % 3], out_hbm.at[wid * BPW + c],
                         sem_o[c % 3])

    def wait_out(c):
        pltpu.make_async_copy(rows_v.at[c % 3], out_hbm.at[wid * BPW + c],
                              sem_o[c % 3]).wait()

    start_gather(0)
    for c in range(BPW):
        if c + 1 < BPW:
            if c >= 2:
                wait_out(c - 2)
            start_gather(c + 1)
        wait_gather(c)
        start_out(c)
    for c in range(BPW - 3, BPW):
        wait_out(c)


def _tr_body(x_ref, o_ref):
    for s in range(SEQ):
        o_ref[s] = jnp.concatenate(
            [x_ref[:, s, :].T, x_ref[:, s + SEQ, :].T], axis=1)


_tr_call = pl.pallas_call(
    _tr_body,
    grid=(8,),
    in_specs=[pl.BlockSpec((B // 2, CH, 128), lambda i: (0, 0, i))],
    out_specs=pl.BlockSpec((SEQ, 128, B), lambda i: (0, i, 0)),
    out_shape=jax.ShapeDtypeStruct((SEQ, N, B), jnp.float32),
)


def kernel(venueid2coor, inputs_poi, poi_distance_matrix):
    table = _recip_call(poi_distance_matrix)
    r3 = _sc_gather(venueid2coor, inputs_poi.reshape(-1), table)
    out2 = _tr_call(r3)
    return out2.transpose(2, 0, 1)
